# baseline (speedup 1.0000x reference)
import functools

import jax
import jax.numpy as jnp
from jax import lax
from jax.experimental import pallas as pl
from jax.experimental.pallas import tpu as pltpu
from jax.experimental.pallas import tpu_sc as plsc

T, D, E, H, O, K = 2048, 768, 8, 1024, 768, 2
P = T * K
BLK = 256
NBLK = P // BLK + E
PCAP = NBLK * BLK
NW = 32
CH_B = P // NW
CH_D = T // NW
_CH = 256


def _gating_body(x_ref, wg_ref, bg_ref, dest_ref, be_ref):
    s = jnp.dot(x_ref[...], wg_ref[...], preferred_element_type=jnp.float32)
    s = s + bg_ref[...]
    ei = lax.broadcasted_iota(jnp.int32, (T, E), 1).astype(jnp.float32)
    m1 = jnp.max(s, axis=1, keepdims=True)
    i1 = jnp.min(jnp.where(s == m1, ei, float(E)), axis=1, keepdims=True)
    oh1 = (ei == i1).astype(jnp.float32)
    s2 = jnp.where(oh1 > 0.0, -jnp.inf, s)
    m2 = jnp.max(s2, axis=1, keepdims=True)
    i2 = jnp.min(jnp.where(s2 == m2, ei, float(E)), axis=1, keepdims=True)
    oh2 = (ei == i2).astype(jnp.float32)

    r = lax.broadcasted_iota(jnp.int32, (_CH, _CH), 0)
    c = lax.broadcasted_iota(jnp.int32, (_CH, _CH), 1)
    tri = (r >= c).astype(jnp.float32)

    def cum(m):
        chunks = []
        off = jnp.zeros((1, E), jnp.float32)
        for i in range(T // _CH):
            blk = jnp.dot(tri, m[i * _CH:(i + 1) * _CH, :],
                          preferred_element_type=jnp.float32) + off
            chunks.append(blk)
            off = blk[_CH - 1:_CH, :]
        return jnp.concatenate(chunks, axis=0)

    c1 = cum(oh1)
    c2 = cum(oh2)
    cnt1 = c1[T - 1:T, :]
    cnt = cnt1 + c2[T - 1:T, :]
    padded = jnp.floor((cnt + float(BLK - 1)) / float(BLK)) * float(BLK)

    re_ = lax.broadcasted_iota(jnp.int32, (E, E), 0)
    ce_ = lax.broadcasted_iota(jnp.int32, (E, E), 1)
    triu = (re_ < ce_).astype(jnp.float32)
    off_e = jnp.dot(padded, triu, preferred_element_type=jnp.float32)

    d1 = jnp.sum(oh1 * (off_e + c1 - 1.0), axis=1, keepdims=True)
    d2 = jnp.sum(oh2 * (off_e + cnt1 + c2 - 1.0), axis=1, keepdims=True)
    dest_ref[:, 0:1] = d1.astype(jnp.int32)
    dest_ref[:, 1:2] = d2.astype(jnp.int32)

    total = jnp.sum(padded, axis=1, keepdims=True)
    bstart = lax.broadcasted_iota(jnp.int32, (NBLK, E), 0).astype(jnp.float32) * float(BLK)
    nge = jnp.sum((bstart >= off_e).astype(jnp.float32), axis=1, keepdims=True)
    start_col = lax.broadcasted_iota(jnp.int32, (NBLK, 1), 0).astype(jnp.float32) * float(BLK)
    be = jnp.where(start_col < total, nge - 1.0, -1.0)

    nonempty = (padded > 0.0).astype(jnp.float32)
    ne_b = jnp.broadcast_to(nonempty, (NBLK, E))
    off_b = jnp.broadcast_to(off_e, (NBLK, E))
    cnt_ne = jnp.sum(ne_b * (off_b <= bstart).astype(jnp.float32),
                     axis=1, keepdims=True)
    c0 = cnt_ne - 1.0
    slot = c0 - 3.0 * jnp.floor(c0 / 3.0)
    first = jnp.sum(ne_b * (off_b == bstart).astype(jnp.float32),
                    axis=1, keepdims=True)
    ei_b = lax.broadcasted_iota(jnp.int32, (NBLK, E), 1).astype(jnp.float32)
    nxt = jnp.min(jnp.where((ne_b > 0.0) & (off_b > bstart), ei_b, float(E)),
                  axis=1, keepdims=True)
    meta = jnp.concatenate([be, slot, first, nxt], axis=1)
    be_ref[...] = meta.astype(jnp.int32)


def _gating(x, wg, bg):
    return pl.pallas_call(
        _gating_body,
        out_shape=(jax.ShapeDtypeStruct((T, K), jnp.int32),
                   jax.ShapeDtypeStruct((NBLK, 4), jnp.int32)),
    )(x, wg, bg)


@functools.cache
def _sc_kernels():
    mesh = plsc.VectorSubcoreMesh(core_axis_name="c", subcore_axis_name="s")

    @functools.partial(
        pl.kernel,
        out_type=jax.ShapeDtypeStruct((16,), jnp.int32),
        mesh=mesh,
        scratch_types=[pltpu.VMEM((16,), jnp.int32)])
    def _warmup(out_hbm, v16):
        wid = lax.axis_index("c") * 16 + lax.axis_index("s")

        @pl.when(wid == 0)
        def _():
            v16[...] = lax.iota(jnp.int32, 16)
            pltpu.sync_copy(v16, out_hbm)

    @functools.partial(
        pl.kernel,
        out_type=jax.ShapeDtypeStruct((PCAP, D), jnp.float32),
        mesh=mesh,
        scratch_types=[pltpu.VMEM((CH_B,), jnp.int32),
                       pltpu.VMEM((CH_B, D), jnp.float32),
                       pltpu.SemaphoreType.DMA])
    def _dispatch(x_hbm, dest_hbm, warm_hbm, xs_hbm, idx_v, rows_v, sem):
        wid = lax.axis_index("c") * 16 + lax.axis_index("s")
        base = wid * CH_B
        tbase = lax.rem(base, T)
        pltpu.sync_copy(dest_hbm.at[pl.ds(base, CH_B)], idx_v)
        pltpu.sync_copy(x_hbm.at[pl.ds(tbase, CH_B)], rows_v)
        pltpu.async_copy(rows_v, xs_hbm.at[idx_v], sem).wait()

    @functools.partial(
        pl.kernel,
        out_type=jax.ShapeDtypeStruct((T, O), jnp.float32),
        mesh=mesh,
        scratch_types=[pltpu.VMEM((CH_D,), jnp.int32),
                       pltpu.VMEM((CH_D,), jnp.int32),
                       pltpu.VMEM((CH_D, O), jnp.float32),
                       pltpu.VMEM((CH_D, O), jnp.float32),
                       pltpu.SemaphoreType.DMA,
                       pltpu.SemaphoreType.DMA])
    def _combine(ys_hbm, dest_hbm, out_hbm, i0_v, i1_v, a_v, b_v, s0, s1):
        wid = lax.axis_index("c") * 16 + lax.axis_index("s")
        base = wid * CH_D
        pltpu.sync_copy(dest_hbm.at[pl.ds(base, CH_D)], i0_v)
        pltpu.sync_copy(dest_hbm.at[pl.ds(T + base, CH_D)], i1_v)
        cp0 = pltpu.async_copy(ys_hbm.at[i0_v], a_v, s0)
        cp1 = pltpu.async_copy(ys_hbm.at[i1_v], b_v, s1)
        cp0.wait()
        cp1.wait()

        def row(rr, carry):
            for cc in range(O // 16):
                sl = pl.ds(cc * 16, 16)
                a_v[rr, sl] = a_v[rr, sl] + b_v[rr, sl]
            return carry

        lax.fori_loop(0, CH_D, row, 0)
        pltpu.sync_copy(a_v, out_hbm.at[pl.ds(base, CH_D)])

    return _warmup, _dispatch, _combine


def _sel(e):
    return jnp.where(e < 0, E - 1, e)


def _expert_body(m_ref, xs_ref, w1_hbm, b1_ref, w2_hbm, b2_ref, out_ref,
                 w1_buf, w2_buf, sems):
    i = pl.program_id(0)
    be = m_ref[i, 0]
    slot = m_ref[i, 1]
    first = m_ref[i, 2]
    nxt = m_ref[i, 3]

    def issue(e, s):
        pltpu.make_async_copy(w1_hbm.at[e], w1_buf.at[s], sems.at[s]).start()
        pltpu.make_async_copy(w2_hbm.at[e], w2_buf.at[s], sems.at[s]).start()

    @pl.when(i == 0)
    def _():
        issue(be, slot)

    @pl.when((first == 1) & (nxt < E))
    def _():
        issue(nxt, lax.rem(slot + 1, 3))

    @pl.when(first == 1)
    def _():
        pltpu.make_async_copy(w1_hbm.at[0], w1_buf.at[slot], sems.at[slot]).wait()
        pltpu.make_async_copy(w2_hbm.at[0], w2_buf.at[slot], sems.at[slot]).wait()

    @pl.when(be >= 0)
    def _():
        h = jnp.dot(xs_ref[...], w1_buf[slot],
                    preferred_element_type=jnp.float32) + b1_ref[0]
        h = jnp.maximum(h, 0.0)
        out_ref[...] = jnp.dot(h, w2_buf[slot],
                               preferred_element_type=jnp.float32) + b2_ref[0]


def _experts(meta, xs, W1, b1, W2, b2):
    grid_spec = pltpu.PrefetchScalarGridSpec(
        num_scalar_prefetch=1,
        grid=(NBLK,),
        in_specs=[
            pl.BlockSpec((BLK, D), lambda i, m: (i, 0)),
            pl.BlockSpec(memory_space=pl.ANY),
            pl.BlockSpec((1, 1, H), lambda i, m: (_sel(m[i, 0]), 0, 0)),
            pl.BlockSpec(memory_space=pl.ANY),
            pl.BlockSpec((1, 1, O), lambda i, m: (_sel(m[i, 0]), 0, 0)),
        ],
        out_specs=pl.BlockSpec((BLK, O), lambda i, m: (i, 0)),
        scratch_shapes=[
            pltpu.VMEM((3, D, H), jnp.float32),
            pltpu.VMEM((3, H, O), jnp.float32),
            pltpu.SemaphoreType.DMA((3,)),
        ],
    )
    return pl.pallas_call(
        _expert_body,
        grid_spec=grid_spec,
        out_shape=jax.ShapeDtypeStruct((PCAP, O), jnp.float32),
    )(meta, xs, W1, b1.reshape(E, 1, H), W2, b2.reshape(E, 1, O))


def kernel(x, Wg, bg, W1, b1, W2, b2):
    warmup, dispatch, combine = _sc_kernels()
    dest, meta = _gating(x, Wg, bg.reshape(1, E))
    dest_km = dest.T.reshape(P)
    warm = warmup()
    xs = dispatch(x, dest_km, warm)
    ys = _experts(meta, xs, W1, b1, W2, b2)
    return combine(ys, dest_km)


# speedup vs baseline: 1.0175x; 1.0175x over previous
import functools

import jax
import jax.numpy as jnp
from jax import lax
from jax.experimental import pallas as pl
from jax.experimental.pallas import tpu as pltpu
from jax.experimental.pallas import tpu_sc as plsc

T, D, E, H, O, K = 2048, 768, 8, 1024, 768, 2
P = T * K
BLK = 256
NBLK = P // BLK + E
PCAP = NBLK * BLK
NW = 32
CH_B = P // NW
CH_D = T // NW
_CH = 256


def _gating_body(x_ref, wg_ref, bg_ref, dest_ref, be_ref):
    s = jnp.dot(x_ref[...], wg_ref[...], preferred_element_type=jnp.float32)
    s = s + bg_ref[...]
    ei = lax.broadcasted_iota(jnp.int32, (T, E), 1).astype(jnp.float32)
    m1 = jnp.max(s, axis=1, keepdims=True)
    i1 = jnp.min(jnp.where(s == m1, ei, float(E)), axis=1, keepdims=True)
    oh1 = (ei == i1).astype(jnp.float32)
    s2 = jnp.where(oh1 > 0.0, -jnp.inf, s)
    m2 = jnp.max(s2, axis=1, keepdims=True)
    i2 = jnp.min(jnp.where(s2 == m2, ei, float(E)), axis=1, keepdims=True)
    oh2 = (ei == i2).astype(jnp.float32)

    r = lax.broadcasted_iota(jnp.int32, (_CH, _CH), 0)
    c = lax.broadcasted_iota(jnp.int32, (_CH, _CH), 1)
    tri = (r >= c).astype(jnp.float32)

    def cum(m):
        chunks = []
        off = jnp.zeros((1, E), jnp.float32)
        for i in range(T // _CH):
            blk = jnp.dot(tri, m[i * _CH:(i + 1) * _CH, :],
                          preferred_element_type=jnp.float32) + off
            chunks.append(blk)
            off = blk[_CH - 1:_CH, :]
        return jnp.concatenate(chunks, axis=0)

    c1 = cum(oh1)
    c2 = cum(oh2)
    cnt1 = c1[T - 1:T, :]
    cnt = cnt1 + c2[T - 1:T, :]
    padded = jnp.floor((cnt + float(BLK - 1)) / float(BLK)) * float(BLK)

    re_ = lax.broadcasted_iota(jnp.int32, (E, E), 0)
    ce_ = lax.broadcasted_iota(jnp.int32, (E, E), 1)
    triu = (re_ < ce_).astype(jnp.float32)
    off_e = jnp.dot(padded, triu, preferred_element_type=jnp.float32)

    d1 = jnp.sum(oh1 * (off_e + c1 - 1.0), axis=1, keepdims=True)
    d2 = jnp.sum(oh2 * (off_e + cnt1 + c2 - 1.0), axis=1, keepdims=True)
    dest_ref[:, 0:1] = d1.astype(jnp.int32)
    dest_ref[:, 1:2] = d2.astype(jnp.int32)

    total = jnp.sum(padded, axis=1, keepdims=True)
    bstart = lax.broadcasted_iota(jnp.int32, (NBLK, E), 0).astype(jnp.float32) * float(BLK)
    nge = jnp.sum((bstart >= off_e).astype(jnp.float32), axis=1, keepdims=True)
    start_col = lax.broadcasted_iota(jnp.int32, (NBLK, 1), 0).astype(jnp.float32) * float(BLK)
    be = jnp.where(start_col < total, nge - 1.0, -1.0)

    nonempty = (padded > 0.0).astype(jnp.float32)
    ne_b = jnp.broadcast_to(nonempty, (NBLK, E))
    off_b = jnp.broadcast_to(off_e, (NBLK, E))
    cnt_ne = jnp.sum(ne_b * (off_b <= bstart).astype(jnp.float32),
                     axis=1, keepdims=True)
    c0 = cnt_ne - 1.0
    slot = c0 - 3.0 * jnp.floor(c0 / 3.0)
    first = jnp.sum(ne_b * (off_b == bstart).astype(jnp.float32),
                    axis=1, keepdims=True)
    ei_b = lax.broadcasted_iota(jnp.int32, (NBLK, E), 1).astype(jnp.float32)
    nxt = jnp.min(jnp.where((ne_b > 0.0) & (off_b > bstart), ei_b, float(E)),
                  axis=1, keepdims=True)
    meta = jnp.concatenate([be, slot, first, nxt], axis=1)
    be_ref[...] = meta.astype(jnp.int32)


def _gating(x, wg, bg):
    return pl.pallas_call(
        _gating_body,
        out_shape=(jax.ShapeDtypeStruct((T, K), jnp.int32),
                   jax.ShapeDtypeStruct((NBLK, 4), jnp.int32)),
    )(x, wg, bg)


@functools.cache
def _sc_kernels():
    mesh = plsc.VectorSubcoreMesh(core_axis_name="c", subcore_axis_name="s")

    @functools.partial(
        pl.kernel,
        out_type=jax.ShapeDtypeStruct((PCAP, D), jnp.float32),
        mesh=mesh,
        scratch_types=[pltpu.VMEM((CH_B,), jnp.int32),
                       pltpu.VMEM((CH_B, D), jnp.float32),
                       pltpu.SemaphoreType.DMA,
                       pltpu.SemaphoreType.DMA])
    def _dispatch(x_hbm, dest_hbm, xs_hbm, idx_v, rows_v, s0, s1):
        wid = lax.axis_index("c") * 16 + lax.axis_index("s")
        base = wid * CH_B
        tbase = lax.rem(base, T)
        cpi = pltpu.async_copy(dest_hbm.at[pl.ds(base, CH_B)], idx_v, s0)
        cpr = pltpu.async_copy(x_hbm.at[pl.ds(tbase, CH_B)], rows_v, s1)
        cpi.wait()
        cpr.wait()
        pltpu.async_copy(rows_v, xs_hbm.at[idx_v], s0).wait()

    @functools.partial(
        pl.kernel,
        out_type=jax.ShapeDtypeStruct((T, O), jnp.float32),
        mesh=mesh,
        scratch_types=[pltpu.VMEM((CH_D,), jnp.int32),
                       pltpu.VMEM((CH_D,), jnp.int32),
                       pltpu.VMEM((CH_D, O), jnp.float32),
                       pltpu.VMEM((CH_D, O), jnp.float32),
                       pltpu.SemaphoreType.DMA,
                       pltpu.SemaphoreType.DMA])
    def _combine(ys_hbm, dest_hbm, out_hbm, i0_v, i1_v, a_v, b_v, s0, s1):
        wid = lax.axis_index("c") * 16 + lax.axis_index("s")
        base = wid * CH_D
        cpi0 = pltpu.async_copy(dest_hbm.at[pl.ds(base, CH_D)], i0_v, s0)
        cpi1 = pltpu.async_copy(dest_hbm.at[pl.ds(T + base, CH_D)], i1_v, s1)
        cpi0.wait()
        cpi1.wait()
        cp0 = pltpu.async_copy(ys_hbm.at[i0_v], a_v, s0)
        cp1 = pltpu.async_copy(ys_hbm.at[i1_v], b_v, s1)
        cp0.wait()
        cp1.wait()

        def row(rr, carry):
            for cc in range(O // 16):
                sl = pl.ds(cc * 16, 16)
                a_v[rr, sl] = a_v[rr, sl] + b_v[rr, sl]
            return carry

        lax.fori_loop(0, CH_D, row, 0)
        pltpu.sync_copy(a_v, out_hbm.at[pl.ds(base, CH_D)])

    return _dispatch, _combine


def _sel(e):
    return jnp.where(e < 0, E - 1, e)


def _expert_body(m_ref, xs_ref, w1_hbm, b1_ref, w2_hbm, b2_ref, out_ref,
                 w1_buf, w2_buf, sems):
    i = pl.program_id(0)
    be = m_ref[i, 0]
    slot = m_ref[i, 1]
    first = m_ref[i, 2]
    nxt = m_ref[i, 3]

    def issue(e, s):
        pltpu.make_async_copy(w1_hbm.at[e], w1_buf.at[s], sems.at[s]).start()
        pltpu.make_async_copy(w2_hbm.at[e], w2_buf.at[s], sems.at[s]).start()

    @pl.when(i == 0)
    def _():
        issue(be, slot)

    @pl.when((first == 1) & (nxt < E))
    def _():
        issue(nxt, lax.rem(slot + 1, 3))

    @pl.when(first == 1)
    def _():
        pltpu.make_async_copy(w1_hbm.at[0], w1_buf.at[slot], sems.at[slot]).wait()
        pltpu.make_async_copy(w2_hbm.at[0], w2_buf.at[slot], sems.at[slot]).wait()

    @pl.when(be >= 0)
    def _():
        h = jnp.dot(xs_ref[...], w1_buf[slot],
                    preferred_element_type=jnp.float32) + b1_ref[0]
        h = jnp.maximum(h, 0.0)
        out_ref[...] = jnp.dot(h, w2_buf[slot],
                               preferred_element_type=jnp.float32) + b2_ref[0]


def _experts(meta, xs, W1, b1, W2, b2):
    grid_spec = pltpu.PrefetchScalarGridSpec(
        num_scalar_prefetch=1,
        grid=(NBLK,),
        in_specs=[
            pl.BlockSpec((BLK, D), lambda i, m: (i, 0)),
            pl.BlockSpec(memory_space=pl.ANY),
            pl.BlockSpec((1, 1, H), lambda i, m: (_sel(m[i, 0]), 0, 0)),
            pl.BlockSpec(memory_space=pl.ANY),
            pl.BlockSpec((1, 1, O), lambda i, m: (_sel(m[i, 0]), 0, 0)),
        ],
        out_specs=pl.BlockSpec((BLK, O), lambda i, m: (i, 0)),
        scratch_shapes=[
            pltpu.VMEM((3, D, H), jnp.float32),
            pltpu.VMEM((3, H, O), jnp.float32),
            pltpu.SemaphoreType.DMA((3,)),
        ],
    )
    return pl.pallas_call(
        _expert_body,
        grid_spec=grid_spec,
        out_shape=jax.ShapeDtypeStruct((PCAP, O), jnp.float32),
    )(meta, xs, W1, b1.reshape(E, 1, H), W2, b2.reshape(E, 1, O))


def kernel(x, Wg, bg, W1, b1, W2, b2):
    dispatch, combine = _sc_kernels()
    dest, meta = _gating(x, Wg, bg.reshape(1, E))
    dest_km = dest.T.reshape(P)
    xs = dispatch(x, dest_km)
    ys = _experts(meta, xs, W1, b1, W2, b2)
    return combine(ys, dest_km)


# speedup vs baseline: 1.0183x; 1.0008x over previous
import functools

import jax
import jax.numpy as jnp
from jax import lax
from jax.experimental import pallas as pl
from jax.experimental.pallas import tpu as pltpu
from jax.experimental.pallas import tpu_sc as plsc

T, D, E, H, O, K = 2048, 768, 8, 1024, 768, 2
P = T * K
BLK = 256
NBLK = P // BLK + E
PCAP = NBLK * BLK
NW = 32
CH_B = P // NW
CH_D = T // NW
_CH = 256


def _gating_body(x_ref, wg_ref, bg_ref, dest_ref, be_ref):
    s = jnp.dot(x_ref[...], wg_ref[...], preferred_element_type=jnp.float32)
    s = s + bg_ref[...]
    ei = lax.broadcasted_iota(jnp.int32, (T, E), 1).astype(jnp.float32)
    m1 = jnp.max(s, axis=1, keepdims=True)
    i1 = jnp.min(jnp.where(s == m1, ei, float(E)), axis=1, keepdims=True)
    oh1 = (ei == i1).astype(jnp.float32)
    s2 = jnp.where(oh1 > 0.0, -jnp.inf, s)
    m2 = jnp.max(s2, axis=1, keepdims=True)
    i2 = jnp.min(jnp.where(s2 == m2, ei, float(E)), axis=1, keepdims=True)
    oh2 = (ei == i2).astype(jnp.float32)

    r = lax.broadcasted_iota(jnp.int32, (_CH, _CH), 0)
    c = lax.broadcasted_iota(jnp.int32, (_CH, _CH), 1)
    tri = (r >= c).astype(jnp.float32)

    def cum(m):
        chunks = []
        off = jnp.zeros((1, E), jnp.float32)
        for i in range(T // _CH):
            blk = jnp.dot(tri, m[i * _CH:(i + 1) * _CH, :],
                          preferred_element_type=jnp.float32) + off
            chunks.append(blk)
            off = blk[_CH - 1:_CH, :]
        return jnp.concatenate(chunks, axis=0)

    c1 = cum(oh1)
    c2 = cum(oh2)
    cnt1 = c1[T - 1:T, :]
    cnt = cnt1 + c2[T - 1:T, :]
    padded = jnp.floor((cnt + float(BLK - 1)) / float(BLK)) * float(BLK)

    re_ = lax.broadcasted_iota(jnp.int32, (E, E), 0)
    ce_ = lax.broadcasted_iota(jnp.int32, (E, E), 1)
    triu = (re_ < ce_).astype(jnp.float32)
    off_e = jnp.dot(padded, triu, preferred_element_type=jnp.float32)

    d1 = jnp.sum(oh1 * (off_e + c1 - 1.0), axis=1, keepdims=True)
    d2 = jnp.sum(oh2 * (off_e + cnt1 + c2 - 1.0), axis=1, keepdims=True)
    dest_ref[:, 0:1] = d1.astype(jnp.int32)
    dest_ref[:, 1:2] = d2.astype(jnp.int32)

    total = jnp.sum(padded, axis=1, keepdims=True)
    bstart = lax.broadcasted_iota(jnp.int32, (NBLK, E), 0).astype(jnp.float32) * float(BLK)
    nge = jnp.sum((bstart >= off_e).astype(jnp.float32), axis=1, keepdims=True)
    start_col = lax.broadcasted_iota(jnp.int32, (NBLK, 1), 0).astype(jnp.float32) * float(BLK)
    be = jnp.where(start_col < total, nge - 1.0, -1.0)

    nonempty = (padded > 0.0).astype(jnp.float32)
    ne_b = jnp.broadcast_to(nonempty, (NBLK, E))
    off_b = jnp.broadcast_to(off_e, (NBLK, E))
    cnt_ne = jnp.sum(ne_b * (off_b <= bstart).astype(jnp.float32),
                     axis=1, keepdims=True)
    c0 = cnt_ne - 1.0
    slot = c0 - 3.0 * jnp.floor(c0 / 3.0)
    first = jnp.sum(ne_b * (off_b == bstart).astype(jnp.float32),
                    axis=1, keepdims=True)
    ei_b = lax.broadcasted_iota(jnp.int32, (NBLK, E), 1).astype(jnp.float32)
    nxt = jnp.min(jnp.where((ne_b > 0.0) & (off_b > bstart), ei_b, float(E)),
                  axis=1, keepdims=True)
    meta = jnp.concatenate([be, slot, first, nxt], axis=1)
    be_ref[...] = meta.astype(jnp.int32)


def _gating(x, wg, bg):
    return pl.pallas_call(
        _gating_body,
        out_shape=(jax.ShapeDtypeStruct((T, K), jnp.int32),
                   jax.ShapeDtypeStruct((NBLK, 4), jnp.int32)),
    )(x, wg, bg)


@functools.cache
def _sc_kernels():
    mesh = plsc.VectorSubcoreMesh(core_axis_name="c", subcore_axis_name="s")

    @functools.partial(
        pl.kernel,
        out_type=jax.ShapeDtypeStruct((PCAP, D), jnp.float32),
        mesh=mesh,
        scratch_types=[pltpu.VMEM((CH_B // 2,), jnp.int32),
                       pltpu.VMEM((CH_B // 2,), jnp.int32),
                       pltpu.VMEM((CH_B // 2, D), jnp.float32),
                       pltpu.VMEM((CH_B // 2, D), jnp.float32),
                       pltpu.SemaphoreType.DMA,
                       pltpu.SemaphoreType.DMA,
                       pltpu.SemaphoreType.DMA,
                       pltpu.SemaphoreType.DMA])
    def _dispatch(x_hbm, dest_hbm, xs_hbm, i0_v, i1_v, r0_v, r1_v,
                  s0, s1, s2, s3):
        wid = lax.axis_index("c") * 16 + lax.axis_index("s")
        base = wid * CH_B
        tbase = lax.rem(base, T)
        hf = CH_B // 2
        cpi0 = pltpu.async_copy(dest_hbm.at[pl.ds(base, hf)], i0_v, s0)
        cpi1 = pltpu.async_copy(dest_hbm.at[pl.ds(base + hf, hf)], i1_v, s1)
        cpr0 = pltpu.async_copy(x_hbm.at[pl.ds(tbase, hf)], r0_v, s2)
        cpr1 = pltpu.async_copy(x_hbm.at[pl.ds(tbase + hf, hf)], r1_v, s3)
        cpi0.wait()
        cpr0.wait()
        sc0 = pltpu.async_copy(r0_v, xs_hbm.at[i0_v], s0)
        cpi1.wait()
        cpr1.wait()
        sc1 = pltpu.async_copy(r1_v, xs_hbm.at[i1_v], s1)
        sc0.wait()
        sc1.wait()

    @functools.partial(
        pl.kernel,
        out_type=jax.ShapeDtypeStruct((T, O), jnp.float32),
        mesh=mesh,
        scratch_types=[pltpu.VMEM((CH_D,), jnp.int32),
                       pltpu.VMEM((CH_D,), jnp.int32),
                       pltpu.VMEM((CH_D, O), jnp.float32),
                       pltpu.VMEM((CH_D, O), jnp.float32),
                       pltpu.SemaphoreType.DMA,
                       pltpu.SemaphoreType.DMA])
    def _combine(ys_hbm, dest_hbm, out_hbm, i0_v, i1_v, a_v, b_v, s0, s1):
        wid = lax.axis_index("c") * 16 + lax.axis_index("s")
        base = wid * CH_D
        cpi0 = pltpu.async_copy(dest_hbm.at[pl.ds(base, CH_D)], i0_v, s0)
        cpi1 = pltpu.async_copy(dest_hbm.at[pl.ds(T + base, CH_D)], i1_v, s1)
        cpi0.wait()
        cpi1.wait()
        cp0 = pltpu.async_copy(ys_hbm.at[i0_v], a_v, s0)
        cp1 = pltpu.async_copy(ys_hbm.at[i1_v], b_v, s1)
        cp0.wait()
        cp1.wait()

        def row(rr, carry):
            for cc in range(O // 16):
                sl = pl.ds(cc * 16, 16)
                a_v[rr, sl] = a_v[rr, sl] + b_v[rr, sl]
            return carry

        lax.fori_loop(0, CH_D, row, 0)
        pltpu.sync_copy(a_v, out_hbm.at[pl.ds(base, CH_D)])

    return _dispatch, _combine


def _sel(e):
    return jnp.where(e < 0, E - 1, e)


def _expert_body(m_ref, xs_ref, w1_hbm, b1_ref, w2_hbm, b2_ref, out_ref,
                 w1_buf, w2_buf, sems):
    i = pl.program_id(0)
    be = m_ref[i, 0]
    slot = m_ref[i, 1]
    first = m_ref[i, 2]
    nxt = m_ref[i, 3]

    def issue(e, s):
        pltpu.make_async_copy(w1_hbm.at[e], w1_buf.at[s], sems.at[s]).start()
        pltpu.make_async_copy(w2_hbm.at[e], w2_buf.at[s], sems.at[s]).start()

    @pl.when(i == 0)
    def _():
        issue(be, slot)

    @pl.when((first == 1) & (nxt < E))
    def _():
        issue(nxt, lax.rem(slot + 1, 3))

    @pl.when(first == 1)
    def _():
        pltpu.make_async_copy(w1_hbm.at[0], w1_buf.at[slot], sems.at[slot]).wait()
        pltpu.make_async_copy(w2_hbm.at[0], w2_buf.at[slot], sems.at[slot]).wait()

    @pl.when(be >= 0)
    def _():
        h = jnp.dot(xs_ref[...], w1_buf[slot],
                    preferred_element_type=jnp.float32) + b1_ref[0]
        h = jnp.maximum(h, 0.0)
        out_ref[...] = jnp.dot(h, w2_buf[slot],
                               preferred_element_type=jnp.float32) + b2_ref[0]


def _experts(meta, xs, W1, b1, W2, b2):
    grid_spec = pltpu.PrefetchScalarGridSpec(
        num_scalar_prefetch=1,
        grid=(NBLK,),
        in_specs=[
            pl.BlockSpec((BLK, D), lambda i, m: (i, 0)),
            pl.BlockSpec(memory_space=pl.ANY),
            pl.BlockSpec((1, 1, H), lambda i, m: (_sel(m[i, 0]), 0, 0)),
            pl.BlockSpec(memory_space=pl.ANY),
            pl.BlockSpec((1, 1, O), lambda i, m: (_sel(m[i, 0]), 0, 0)),
        ],
        out_specs=pl.BlockSpec((BLK, O), lambda i, m: (i, 0)),
        scratch_shapes=[
            pltpu.VMEM((3, D, H), jnp.float32),
            pltpu.VMEM((3, H, O), jnp.float32),
            pltpu.SemaphoreType.DMA((3,)),
        ],
    )
    return pl.pallas_call(
        _expert_body,
        grid_spec=grid_spec,
        out_shape=jax.ShapeDtypeStruct((PCAP, O), jnp.float32),
    )(meta, xs, W1, b1.reshape(E, 1, H), W2, b2.reshape(E, 1, O))


def kernel(x, Wg, bg, W1, b1, W2, b2):
    dispatch, combine = _sc_kernels()
    dest, meta = _gating(x, Wg, bg.reshape(1, E))
    dest_km = dest.T.reshape(P)
    xs = dispatch(x, dest_km)
    ys = _experts(meta, xs, W1, b1, W2, b2)
    return combine(ys, dest_km)
